# baseline (speedup 1.0000x reference)
import functools

import jax
import jax.numpy as jnp
from jax import lax
from jax.experimental import pallas as pl
from jax.experimental.pallas import tpu as pltpu
from jax.experimental.pallas import tpu_sc as plsc

N = 10000
D = 128
HALF = 64
E = 320000
CHUNK = 128
K0_CHUNKS = 80
E_PAD = 32 * K0_CHUNKS * CHUNK
K2_CHUNKS = 2 * K0_CHUNKS
NBUF = 4
DUMMY = N
N_ACC = 10112
ACC_SLC = N_ACC // 16
RBLK = 1000


def _mesh():
    return plsc.VectorSubcoreMesh(core_axis_name="c", subcore_axis_name="s",
                                  num_cores=2, num_subcores=16)


def _zero_rows(ref, nrows, ncols16):
    zero = jnp.zeros((16,), jnp.float32)

    def body(t, _):
        i = t // ncols16
        k = t % ncols16
        ref[i, pl.ds(k * 16, 16)] = zero
        return 0

    lax.fori_loop(0, nrows * ncols16, body, 0)


def _copy_zero_slice(zsrc, shared, base, nrows):
    full, rem = nrows // CHUNK, nrows % CHUNK
    for k in range(full):
        pltpu.sync_copy(zsrc, shared.at[pl.ds(base + k * CHUNK, CHUNK)])
    if rem:
        pltpu.sync_copy(zsrc.at[pl.ds(0, rem)],
                        shared.at[pl.ds(base + full * CHUNK, rem)])


def _deg_body(dst_hbm, deg_out, dst_v, ones_v, deg_sh):
    c = lax.axis_index("c")
    s = lax.axis_index("s")
    tid = c * 16 + s
    pltpu.sync_copy(dst_hbm.at[tid], dst_v)

    _zero_rows(ones_v, CHUNK, 1)
    _copy_zero_slice(ones_v, deg_sh, ACC_SLC * s, ACC_SLC)

    one = jnp.full((16,), 1.0, jnp.float32)

    def fill(i, _):
        ones_v[i, :] = one
        return 0

    lax.fori_loop(0, CHUNK, fill, 0)
    plsc.subcore_barrier()

    def body(j, _):
        pltpu.sync_copy(ones_v, deg_sh.at[dst_v.at[j]], add=True)
        return 0

    lax.fori_loop(0, K0_CHUNKS, body, 0)
    plsc.subcore_barrier()
    pltpu.sync_copy(deg_sh.at[pl.ds(ACC_SLC * s, ACC_SLC)],
                    deg_out.at[c, pl.ds(ACC_SLC * s, ACC_SLC)])


def _gather_scatter_body(hs_hbm, src_hbm, dst_hbm, acc_out,
                           src_v, dst_v, rows, acc_sh, gsems):
    c = lax.axis_index("c")
    s = lax.axis_index("s")
    hs_tbl = hs_hbm.at[c]

    pltpu.sync_copy(src_hbm.at[2 * s], src_v.at[pl.ds(0, K0_CHUNKS)])
    pltpu.sync_copy(src_hbm.at[2 * s + 1], src_v.at[pl.ds(K0_CHUNKS, K0_CHUNKS)])
    pltpu.sync_copy(dst_hbm.at[2 * s], dst_v.at[pl.ds(0, K0_CHUNKS)])
    pltpu.sync_copy(dst_hbm.at[2 * s + 1], dst_v.at[pl.ds(K0_CHUNKS, K0_CHUNKS)])

    _zero_rows(rows[0], CHUNK, HALF // 16)
    _copy_zero_slice(rows[0], acc_sh, ACC_SLC * s, ACC_SLC)
    plsc.subcore_barrier()

    def group(g, _):
        j0 = g * NBUF
        for b in range(NBUF):
            pltpu.async_copy(hs_tbl.at[src_v.at[j0 + b]], rows[b], gsems.at[b])
        for b in range(NBUF):
            pltpu.make_async_copy(hs_tbl.at[src_v.at[j0 + b]], rows[b],
                                  gsems.at[b]).wait()
        for b in range(NBUF):
            pltpu.sync_copy(rows[b], acc_sh.at[dst_v.at[j0 + b]], add=True)
        return 0

    lax.fori_loop(0, K2_CHUNKS // NBUF, group, 0)
    plsc.subcore_barrier()
    pltpu.sync_copy(acc_sh.at[pl.ds(ACC_SLC * s, ACC_SLC)],
                    acc_out.at[c, pl.ds(ACC_SLC * s, ACC_SLC)])


@functools.cache
def _sc_kernels():
    deg_k = pl.kernel(
        _deg_body,
        out_type=jax.ShapeDtypeStruct((2, N_ACC, 16), jnp.float32),
        mesh=_mesh(),
        scratch_types=[
            pltpu.VMEM((K0_CHUNKS, CHUNK), jnp.int32),
            pltpu.VMEM((CHUNK, 16), jnp.float32),
            pltpu.VMEM_SHARED((N_ACC, 16), jnp.float32),
        ],
    )
    gs_k = pl.kernel(
        _gather_scatter_body,
        out_type=jax.ShapeDtypeStruct((2, N_ACC, HALF), jnp.float32),
        mesh=_mesh(),
        scratch_types=[
            pltpu.VMEM((K2_CHUNKS, CHUNK), jnp.int32),
            pltpu.VMEM((K2_CHUNKS, CHUNK), jnp.int32),
            [pltpu.VMEM((CHUNK, HALF), jnp.float32)] * NBUF,
            pltpu.VMEM_SHARED((N_ACC, HALF), jnp.float32),
            pltpu.SemaphoreType.DMA((NBUF,)),
        ],
        compiler_params=pltpu.CompilerParams(use_tc_tiling_on_sc=False),
    )
    return deg_k, gs_k


def _dinv_from(deg_ref):
    deg = deg_ref[0, :, 0:1] + deg_ref[1, :, 0:1] + 1.0
    return lax.rsqrt(deg)


def _layer1_body(deg_ref, x_ref, w_ref, hs_ref):
    dinv = _dinv_from(deg_ref)
    h = jnp.dot(x_ref[...], w_ref[...], preferred_element_type=jnp.float32)
    hs = h * dinv
    hs_ref[0] = hs[:, :HALF]
    hs_ref[1] = hs[:, HALF:]


def _layer2_body(deg_ref, acc_ref, hs1_ref, b1_ref, w_ref, hs2_ref):
    dinv = _dinv_from(deg_ref)
    z0 = dinv * (acc_ref[0] + hs1_ref[0])
    z1 = dinv * (acc_ref[1] + hs1_ref[1])
    z = jnp.concatenate([z0, z1], axis=1) + b1_ref[...]
    z = jnp.maximum(z, 0.0)
    h2 = jnp.dot(z, w_ref[...], preferred_element_type=jnp.float32)
    hs2 = h2 * dinv
    hs2_ref[0] = hs2[:, :HALF]
    hs2_ref[1] = hs2[:, HALF:]


def _final_body(deg_ref, acc_ref, hs2_ref, b2_ref, out_ref):
    dinv = _dinv_from(deg_ref)
    o0 = dinv * (acc_ref[0] + hs2_ref[0])
    o1 = dinv * (acc_ref[1] + hs2_ref[1])
    out_ref[...] = jnp.concatenate([o0, o1], axis=1) + b2_ref[...]


_DEG_SPEC = pl.BlockSpec((2, RBLK, 16), lambda i: (0, i, 0))
_SPLIT_SPEC = pl.BlockSpec((2, RBLK, HALF), lambda i: (0, i, 0))
_FULL_SPEC = pl.BlockSpec((RBLK, D), lambda i: (i, 0))
_W_SPEC = pl.BlockSpec((D, D), lambda i: (0, 0))
_B_SPEC = pl.BlockSpec((1, D), lambda i: (0, 0))

_layer1 = pl.pallas_call(
    _layer1_body,
    grid=(N // RBLK,),
    in_specs=[_DEG_SPEC, _FULL_SPEC, _W_SPEC],
    out_specs=_SPLIT_SPEC,
    out_shape=jax.ShapeDtypeStruct((2, N_ACC, HALF), jnp.float32),
)

_layer2 = pl.pallas_call(
    _layer2_body,
    grid=(N // RBLK,),
    in_specs=[_DEG_SPEC, _SPLIT_SPEC, _SPLIT_SPEC, _B_SPEC, _W_SPEC],
    out_specs=_SPLIT_SPEC,
    out_shape=jax.ShapeDtypeStruct((2, N_ACC, HALF), jnp.float32),
)

_final = pl.pallas_call(
    _final_body,
    grid=(N // RBLK,),
    in_specs=[_DEG_SPEC, _SPLIT_SPEC, _SPLIT_SPEC, _B_SPEC],
    out_specs=_FULL_SPEC,
    out_shape=jax.ShapeDtypeStruct((N, D), jnp.float32),
)


@jax.jit
def _kernel_impl(x, edge_index, W1, b1, W2, b2):
    deg_kernel, gather_scatter_kernel = _sc_kernels()
    src = edge_index[0].astype(jnp.int32)
    dst = edge_index[1].astype(jnp.int32)
    pad = E_PAD - E
    src = jnp.concatenate([src, jnp.zeros((pad,), jnp.int32)])
    dst = jnp.concatenate([dst, jnp.full((pad,), DUMMY, jnp.int32)])
    src = src.reshape(32, K0_CHUNKS, CHUNK)
    dst = dst.reshape(32, K0_CHUNKS, CHUNK)

    deg = deg_kernel(dst)
    hs1 = _layer1(deg, x, W1)
    acc1 = gather_scatter_kernel(hs1, src, dst)
    hs2 = _layer2(deg, acc1, hs1, b1.reshape(1, D), W2)
    acc2 = gather_scatter_kernel(hs2, src, dst)
    return _final(deg, acc2, hs2, b2.reshape(1, D))


def kernel(x, edge_index, W1, b1, W2, b2):
    return _kernel_impl(x, edge_index, W1, b1, W2, b2)


# speedup vs baseline: 1.0188x; 1.0188x over previous
import functools

import jax
import jax.numpy as jnp
from jax import lax
from jax.experimental import pallas as pl
from jax.experimental.pallas import tpu as pltpu
from jax.experimental.pallas import tpu_sc as plsc

N = 10000
D = 128
HALF = 64
E = 320000
CHUNK = 128
K0_CHUNKS = 80
E_PAD = 32 * K0_CHUNKS * CHUNK
K2_CHUNKS = 2 * K0_CHUNKS
NBUF = 5
DUMMY = N
N_ACC = 10112
ACC_SLC = N_ACC // 16
RBLK = 1000


def _mesh():
    return plsc.VectorSubcoreMesh(core_axis_name="c", subcore_axis_name="s",
                                  num_cores=2, num_subcores=16)


def _zero_rows(ref, nrows, ncols16):
    zero = jnp.zeros((16,), jnp.float32)

    def body(t, _):
        i = t // ncols16
        k = t % ncols16
        ref[i, pl.ds(k * 16, 16)] = zero
        return 0

    lax.fori_loop(0, nrows * ncols16, body, 0)


def _copy_zero_slice(zsrc, shared, base, nrows):
    full, rem = nrows // CHUNK, nrows % CHUNK
    for k in range(full):
        pltpu.sync_copy(zsrc, shared.at[pl.ds(base + k * CHUNK, CHUNK)])
    if rem:
        pltpu.sync_copy(zsrc.at[pl.ds(0, rem)],
                        shared.at[pl.ds(base + full * CHUNK, rem)])


def _deg_body(dst_hbm, deg_out, dst_v, ones_v, deg_sh):
    c = lax.axis_index("c")
    s = lax.axis_index("s")
    tid = c * 16 + s
    pltpu.sync_copy(dst_hbm.at[tid], dst_v)

    _zero_rows(ones_v, CHUNK, 1)
    _copy_zero_slice(ones_v, deg_sh, ACC_SLC * s, ACC_SLC)

    one = jnp.full((16,), 1.0, jnp.float32)

    def fill(i, _):
        ones_v[i, :] = one
        return 0

    lax.fori_loop(0, CHUNK, fill, 0)
    plsc.subcore_barrier()

    def body(j, _):
        pltpu.sync_copy(ones_v, deg_sh.at[dst_v.at[j]], add=True)
        return 0

    lax.fori_loop(0, K0_CHUNKS, body, 0)
    plsc.subcore_barrier()
    pltpu.sync_copy(deg_sh.at[pl.ds(ACC_SLC * s, ACC_SLC)],
                    deg_out.at[c, pl.ds(ACC_SLC * s, ACC_SLC)])


def _gather_scatter_body(hs_hbm, src_hbm, dst_hbm, acc_out,
                           src_v, dst_v, rows, acc_sh, gsem, ssem):
    c = lax.axis_index("c")
    s = lax.axis_index("s")
    hs_tbl = hs_hbm.at[c]

    pltpu.sync_copy(src_hbm.at[2 * s], src_v.at[pl.ds(0, K0_CHUNKS)])
    pltpu.sync_copy(src_hbm.at[2 * s + 1], src_v.at[pl.ds(K0_CHUNKS, K0_CHUNKS)])
    pltpu.sync_copy(dst_hbm.at[2 * s], dst_v.at[pl.ds(0, K0_CHUNKS)])
    pltpu.sync_copy(dst_hbm.at[2 * s + 1], dst_v.at[pl.ds(K0_CHUNKS, K0_CHUNKS)])

    _zero_rows(rows[0], CHUNK, HALF // 16)
    _copy_zero_slice(rows[0], acc_sh, ACC_SLC * s, ACC_SLC)
    plsc.subcore_barrier()

    def group(g, _):
        j0 = g * NBUF
        for b in range(NBUF):
            pltpu.async_copy(hs_tbl.at[src_v.at[j0 + b]], rows[b], gsem)
        for b in range(NBUF):
            pltpu.make_async_copy(hs_tbl.at[src_v.at[j0 + b]], rows[b],
                                  gsem).wait()
        for b in range(NBUF):
            pltpu.async_copy(rows[b], acc_sh.at[dst_v.at[j0 + b]], ssem,
                             add=True)
        for b in range(NBUF):
            pltpu.make_async_copy(rows[b], acc_sh.at[dst_v.at[j0 + b]],
                                  ssem).wait()
        return 0

    lax.fori_loop(0, K2_CHUNKS // NBUF, group, 0)
    plsc.subcore_barrier()
    pltpu.sync_copy(acc_sh.at[pl.ds(ACC_SLC * s, ACC_SLC)],
                    acc_out.at[c, pl.ds(ACC_SLC * s, ACC_SLC)])


@functools.cache
def _sc_kernels():
    deg_k = pl.kernel(
        _deg_body,
        out_type=jax.ShapeDtypeStruct((2, N_ACC, 16), jnp.float32),
        mesh=_mesh(),
        scratch_types=[
            pltpu.VMEM((K0_CHUNKS, CHUNK), jnp.int32),
            pltpu.VMEM((CHUNK, 16), jnp.float32),
            pltpu.VMEM_SHARED((N_ACC, 16), jnp.float32),
        ],
    )
    gs_k = pl.kernel(
        _gather_scatter_body,
        out_type=jax.ShapeDtypeStruct((2, N_ACC, HALF), jnp.float32),
        mesh=_mesh(),
        scratch_types=[
            pltpu.VMEM((K2_CHUNKS, CHUNK), jnp.int32),
            pltpu.VMEM((K2_CHUNKS, CHUNK), jnp.int32),
            [pltpu.VMEM((CHUNK, HALF), jnp.float32)] * NBUF,
            pltpu.VMEM_SHARED((N_ACC, HALF), jnp.float32),
            pltpu.SemaphoreType.DMA,
            pltpu.SemaphoreType.DMA,
        ],
        compiler_params=pltpu.CompilerParams(use_tc_tiling_on_sc=False),
    )
    return deg_k, gs_k


def _dinv_from(deg_ref):
    deg = deg_ref[0, :, 0:1] + deg_ref[1, :, 0:1] + 1.0
    return lax.rsqrt(deg)


def _layer1_body(deg_ref, x_ref, w_ref, hs_ref):
    dinv = _dinv_from(deg_ref)
    h = jnp.dot(x_ref[...], w_ref[...], preferred_element_type=jnp.float32)
    hs = h * dinv
    hs_ref[0] = hs[:, :HALF]
    hs_ref[1] = hs[:, HALF:]


def _layer2_body(deg_ref, acc_ref, hs1_ref, b1_ref, w_ref, hs2_ref):
    dinv = _dinv_from(deg_ref)
    z0 = dinv * (acc_ref[0] + hs1_ref[0])
    z1 = dinv * (acc_ref[1] + hs1_ref[1])
    z = jnp.concatenate([z0, z1], axis=1) + b1_ref[...]
    z = jnp.maximum(z, 0.0)
    h2 = jnp.dot(z, w_ref[...], preferred_element_type=jnp.float32)
    hs2 = h2 * dinv
    hs2_ref[0] = hs2[:, :HALF]
    hs2_ref[1] = hs2[:, HALF:]


def _final_body(deg_ref, acc_ref, hs2_ref, b2_ref, out_ref):
    dinv = _dinv_from(deg_ref)
    o0 = dinv * (acc_ref[0] + hs2_ref[0])
    o1 = dinv * (acc_ref[1] + hs2_ref[1])
    out_ref[...] = jnp.concatenate([o0, o1], axis=1) + b2_ref[...]


_DEG_SPEC = pl.BlockSpec((2, RBLK, 16), lambda i: (0, i, 0))
_SPLIT_SPEC = pl.BlockSpec((2, RBLK, HALF), lambda i: (0, i, 0))
_FULL_SPEC = pl.BlockSpec((RBLK, D), lambda i: (i, 0))
_W_SPEC = pl.BlockSpec((D, D), lambda i: (0, 0))
_B_SPEC = pl.BlockSpec((1, D), lambda i: (0, 0))

_layer1 = pl.pallas_call(
    _layer1_body,
    grid=(N // RBLK,),
    in_specs=[_DEG_SPEC, _FULL_SPEC, _W_SPEC],
    out_specs=_SPLIT_SPEC,
    out_shape=jax.ShapeDtypeStruct((2, N_ACC, HALF), jnp.float32),
)

_layer2 = pl.pallas_call(
    _layer2_body,
    grid=(N // RBLK,),
    in_specs=[_DEG_SPEC, _SPLIT_SPEC, _SPLIT_SPEC, _B_SPEC, _W_SPEC],
    out_specs=_SPLIT_SPEC,
    out_shape=jax.ShapeDtypeStruct((2, N_ACC, HALF), jnp.float32),
)

_final = pl.pallas_call(
    _final_body,
    grid=(N // RBLK,),
    in_specs=[_DEG_SPEC, _SPLIT_SPEC, _SPLIT_SPEC, _B_SPEC],
    out_specs=_FULL_SPEC,
    out_shape=jax.ShapeDtypeStruct((N, D), jnp.float32),
)


@jax.jit
def _kernel_impl(x, edge_index, W1, b1, W2, b2):
    deg_kernel, gather_scatter_kernel = _sc_kernels()
    src = edge_index[0].astype(jnp.int32)
    dst = edge_index[1].astype(jnp.int32)
    pad = E_PAD - E
    src = jnp.concatenate([src, jnp.zeros((pad,), jnp.int32)])
    dst = jnp.concatenate([dst, jnp.full((pad,), DUMMY, jnp.int32)])
    src = src.reshape(32, K0_CHUNKS, CHUNK)
    dst = dst.reshape(32, K0_CHUNKS, CHUNK)

    deg = deg_kernel(dst)
    hs1 = _layer1(deg, x, W1)
    acc1 = gather_scatter_kernel(hs1, src, dst)
    hs2 = _layer2(deg, acc1, hs1, b1.reshape(1, D), W2)
    acc2 = gather_scatter_kernel(hs2, src, dst)
    return _final(deg, acc2, hs2, b2.reshape(1, D))


def kernel(x, edge_index, W1, b1, W2, b2):
    return _kernel_impl(x, edge_index, W1, b1, W2, b2)


# speedup vs baseline: 1.1099x; 1.0894x over previous
import functools

import jax
import jax.numpy as jnp
from jax import lax
from jax.experimental import pallas as pl
from jax.experimental.pallas import tpu as pltpu
from jax.experimental.pallas import tpu_sc as plsc

N = 10000
D = 128
HALF = 64
E = 320000
CHUNK = 128
K0_CHUNKS = 80
E_PAD = 32 * K0_CHUNKS * CHUNK
K2_CHUNKS = 2 * K0_CHUNKS
NBUF = 2
DUMMY = N
N_ACC = 10112
ACC_SLC = N_ACC // 16
RBLK = 1000


def _mesh():
    return plsc.VectorSubcoreMesh(core_axis_name="c", subcore_axis_name="s",
                                  num_cores=2, num_subcores=16)


def _zero_rows(ref, nrows, ncols16):
    zero = jnp.zeros((16,), jnp.float32)

    def body(t, _):
        i = t // ncols16
        k = t % ncols16
        ref[i, pl.ds(k * 16, 16)] = zero
        return 0

    lax.fori_loop(0, nrows * ncols16, body, 0)


def _copy_zero_slice(zsrc, shared, base, nrows):
    full, rem = nrows // CHUNK, nrows % CHUNK
    for k in range(full):
        pltpu.sync_copy(zsrc, shared.at[pl.ds(base + k * CHUNK, CHUNK)])
    if rem:
        pltpu.sync_copy(zsrc.at[pl.ds(0, rem)],
                        shared.at[pl.ds(base + full * CHUNK, rem)])


def _deg_body(dst_hbm, deg_out, dst_v, ones_v, deg_sh):
    c = lax.axis_index("c")
    s = lax.axis_index("s")
    tid = c * 16 + s
    pltpu.sync_copy(dst_hbm.at[tid], dst_v)

    _zero_rows(ones_v, CHUNK, 1)
    _copy_zero_slice(ones_v, deg_sh, ACC_SLC * s, ACC_SLC)

    one = jnp.full((16,), 1.0, jnp.float32)

    def fill(i, _):
        ones_v[i, :] = one
        return 0

    lax.fori_loop(0, CHUNK, fill, 0)
    plsc.subcore_barrier()

    def body(j, _):
        pltpu.sync_copy(ones_v, deg_sh.at[dst_v.at[j]], add=True)
        return 0

    lax.fori_loop(0, K0_CHUNKS, body, 0)
    plsc.subcore_barrier()
    pltpu.sync_copy(deg_sh.at[pl.ds(ACC_SLC * s, ACC_SLC)],
                    deg_out.at[c, pl.ds(ACC_SLC * s, ACC_SLC)])


def _gather_scatter_body(hs_hbm, src_hbm, dst_hbm, acc_out,
                           src_v, dst_v, rows, acc_sh, gsem, ssem):
    c = lax.axis_index("c")
    s = lax.axis_index("s")
    hs_tbl = hs_hbm.at[c]

    pltpu.sync_copy(src_hbm.at[2 * s], src_v.at[pl.ds(0, K0_CHUNKS)])
    pltpu.sync_copy(src_hbm.at[2 * s + 1], src_v.at[pl.ds(K0_CHUNKS, K0_CHUNKS)])
    pltpu.sync_copy(dst_hbm.at[2 * s], dst_v.at[pl.ds(0, K0_CHUNKS)])
    pltpu.sync_copy(dst_hbm.at[2 * s + 1], dst_v.at[pl.ds(K0_CHUNKS, K0_CHUNKS)])

    _zero_rows(rows[0], CHUNK, HALF // 16)
    _copy_zero_slice(rows[0], acc_sh, ACC_SLC * s, ACC_SLC)
    plsc.subcore_barrier()

    sems = (gsem, ssem)
    pltpu.async_copy(hs_tbl.at[src_v.at[0]], rows[0], sems[0])

    def step(i, _):
        for b in range(2):
            j = 2 * i + b
            nxt = 1 - b

            @pl.when(j + 1 < K2_CHUNKS)
            def _():
                pltpu.async_copy(hs_tbl.at[src_v.at[j + 1]], rows[nxt],
                                 sems[nxt])

            pltpu.make_async_copy(hs_tbl.at[src_v.at[j]], rows[b],
                                  sems[b]).wait()
            pltpu.sync_copy(rows[b], acc_sh.at[dst_v.at[j]], add=True)
        return 0

    lax.fori_loop(0, K2_CHUNKS // 2, step, 0)
    plsc.subcore_barrier()
    pltpu.sync_copy(acc_sh.at[pl.ds(ACC_SLC * s, ACC_SLC)],
                    acc_out.at[c, pl.ds(ACC_SLC * s, ACC_SLC)])


@functools.cache
def _sc_kernels():
    deg_k = pl.kernel(
        _deg_body,
        out_type=jax.ShapeDtypeStruct((2, N_ACC, 16), jnp.float32),
        mesh=_mesh(),
        scratch_types=[
            pltpu.VMEM((K0_CHUNKS, CHUNK), jnp.int32),
            pltpu.VMEM((CHUNK, 16), jnp.float32),
            pltpu.VMEM_SHARED((N_ACC, 16), jnp.float32),
        ],
    )
    gs_k = pl.kernel(
        _gather_scatter_body,
        out_type=jax.ShapeDtypeStruct((2, N_ACC, HALF), jnp.float32),
        mesh=_mesh(),
        scratch_types=[
            pltpu.VMEM((K2_CHUNKS, CHUNK), jnp.int32),
            pltpu.VMEM((K2_CHUNKS, CHUNK), jnp.int32),
            [pltpu.VMEM((CHUNK, HALF), jnp.float32)] * NBUF,
            pltpu.VMEM_SHARED((N_ACC, HALF), jnp.float32),
            pltpu.SemaphoreType.DMA,
            pltpu.SemaphoreType.DMA,
        ],
        compiler_params=pltpu.CompilerParams(use_tc_tiling_on_sc=False),
    )
    return deg_k, gs_k


def _dinv_from(deg_ref):
    deg = deg_ref[0, :, 0:1] + deg_ref[1, :, 0:1] + 1.0
    return lax.rsqrt(deg)


def _layer1_body(deg_ref, x_ref, w_ref, hs_ref):
    dinv = _dinv_from(deg_ref)
    h = jnp.dot(x_ref[...], w_ref[...], preferred_element_type=jnp.float32)
    hs = h * dinv
    hs_ref[0] = hs[:, :HALF]
    hs_ref[1] = hs[:, HALF:]


def _layer2_body(deg_ref, acc_ref, hs1_ref, b1_ref, w_ref, hs2_ref):
    dinv = _dinv_from(deg_ref)
    z0 = dinv * (acc_ref[0] + hs1_ref[0])
    z1 = dinv * (acc_ref[1] + hs1_ref[1])
    z = jnp.concatenate([z0, z1], axis=1) + b1_ref[...]
    z = jnp.maximum(z, 0.0)
    h2 = jnp.dot(z, w_ref[...], preferred_element_type=jnp.float32)
    hs2 = h2 * dinv
    hs2_ref[0] = hs2[:, :HALF]
    hs2_ref[1] = hs2[:, HALF:]


def _final_body(deg_ref, acc_ref, hs2_ref, b2_ref, out_ref):
    dinv = _dinv_from(deg_ref)
    o0 = dinv * (acc_ref[0] + hs2_ref[0])
    o1 = dinv * (acc_ref[1] + hs2_ref[1])
    out_ref[...] = jnp.concatenate([o0, o1], axis=1) + b2_ref[...]


_DEG_SPEC = pl.BlockSpec((2, RBLK, 16), lambda i: (0, i, 0))
_SPLIT_SPEC = pl.BlockSpec((2, RBLK, HALF), lambda i: (0, i, 0))
_FULL_SPEC = pl.BlockSpec((RBLK, D), lambda i: (i, 0))
_W_SPEC = pl.BlockSpec((D, D), lambda i: (0, 0))
_B_SPEC = pl.BlockSpec((1, D), lambda i: (0, 0))

_layer1 = pl.pallas_call(
    _layer1_body,
    grid=(N // RBLK,),
    in_specs=[_DEG_SPEC, _FULL_SPEC, _W_SPEC],
    out_specs=_SPLIT_SPEC,
    out_shape=jax.ShapeDtypeStruct((2, N_ACC, HALF), jnp.float32),
)

_layer2 = pl.pallas_call(
    _layer2_body,
    grid=(N // RBLK,),
    in_specs=[_DEG_SPEC, _SPLIT_SPEC, _SPLIT_SPEC, _B_SPEC, _W_SPEC],
    out_specs=_SPLIT_SPEC,
    out_shape=jax.ShapeDtypeStruct((2, N_ACC, HALF), jnp.float32),
)

_final = pl.pallas_call(
    _final_body,
    grid=(N // RBLK,),
    in_specs=[_DEG_SPEC, _SPLIT_SPEC, _SPLIT_SPEC, _B_SPEC],
    out_specs=_FULL_SPEC,
    out_shape=jax.ShapeDtypeStruct((N, D), jnp.float32),
)


@jax.jit
def _kernel_impl(x, edge_index, W1, b1, W2, b2):
    deg_kernel, gather_scatter_kernel = _sc_kernels()
    src = edge_index[0].astype(jnp.int32)
    dst = edge_index[1].astype(jnp.int32)
    pad = E_PAD - E
    src = jnp.concatenate([src, jnp.zeros((pad,), jnp.int32)])
    dst = jnp.concatenate([dst, jnp.full((pad,), DUMMY, jnp.int32)])
    src = src.reshape(32, K0_CHUNKS, CHUNK)
    dst = dst.reshape(32, K0_CHUNKS, CHUNK)

    deg = deg_kernel(dst)
    hs1 = _layer1(deg, x, W1)
    acc1 = gather_scatter_kernel(hs1, src, dst)
    hs2 = _layer2(deg, acc1, hs1, b1.reshape(1, D), W2)
    acc2 = gather_scatter_kernel(hs2, src, dst)
    return _final(deg, acc2, hs2, b2.reshape(1, D))


def kernel(x, edge_index, W1, b1, W2, b2):
    return _kernel_impl(x, edge_index, W1, b1, W2, b2)


# speedup vs baseline: 1.5025x; 1.3537x over previous
import functools

import jax
import jax.numpy as jnp
from jax import lax
from jax.experimental import pallas as pl
from jax.experimental.pallas import tpu as pltpu
from jax.experimental.pallas import tpu_sc as plsc

N = 10000
D = 128
HALF = 64
E = 320000
CHUNK = 128
K0_CHUNKS = 79
E_PAD = 32 * K0_CHUNKS * CHUNK
K2_CHUNKS = 2 * K0_CHUNKS
NBUF = 2
DUMMY = N
N_ACC = 10112
ACC_SLC = N_ACC // 16
RBLK = 1000


def _mesh():
    return plsc.VectorSubcoreMesh(core_axis_name="c", subcore_axis_name="s",
                                  num_cores=2, num_subcores=16)


def _zero_rows(ref, nrows, ncols16):
    zero = jnp.zeros((16,), jnp.float32)

    def body(t, _):
        i = t // ncols16
        k = t % ncols16
        ref[i, pl.ds(k * 16, 16)] = zero
        return 0

    lax.fori_loop(0, nrows * ncols16, body, 0)


def _copy_zero_slice(zsrc, shared, base, nrows):
    full, rem = nrows // CHUNK, nrows % CHUNK
    for k in range(full):
        pltpu.sync_copy(zsrc, shared.at[pl.ds(base + k * CHUNK, CHUNK)])
    if rem:
        pltpu.sync_copy(zsrc.at[pl.ds(0, rem)],
                        shared.at[pl.ds(base + full * CHUNK, rem)])


def _deg_body(dst_hbm, deg_out, dst_v, ones_v, deg_sh):
    c = lax.axis_index("c")
    s = lax.axis_index("s")
    tid = c * 16 + s
    pltpu.sync_copy(dst_hbm.at[tid], dst_v)

    _zero_rows(ones_v, CHUNK, 1)
    _copy_zero_slice(ones_v, deg_sh, ACC_SLC * s, ACC_SLC)

    one = jnp.full((16,), 1.0, jnp.float32)

    def fill(i, _):
        ones_v[i, :] = one
        return 0

    lax.fori_loop(0, CHUNK, fill, 0)
    plsc.subcore_barrier()

    def body(j, _):
        pltpu.sync_copy(ones_v, deg_sh.at[dst_v.at[j]], add=True)
        return 0

    lax.fori_loop(0, K0_CHUNKS, body, 0)
    plsc.subcore_barrier()
    pltpu.sync_copy(deg_sh.at[pl.ds(ACC_SLC * s, ACC_SLC)],
                    deg_out.at[c, pl.ds(ACC_SLC * s, ACC_SLC)])


def _gather_scatter_body(hs_hbm, src_hbm, dst_hbm, acc_out,
                           src_v, dst_v, rows0, rows1, acc_sh, gsem, ssem):
    rows = (rows0, rows1)
    c = lax.axis_index("c")
    s = lax.axis_index("s")
    hs_tbl = hs_hbm.at[c]

    pltpu.sync_copy(src_hbm.at[2 * s], src_v.at[pl.ds(0, K0_CHUNKS)])
    pltpu.sync_copy(src_hbm.at[2 * s + 1], src_v.at[pl.ds(K0_CHUNKS, K0_CHUNKS)])
    pltpu.sync_copy(dst_hbm.at[2 * s], dst_v.at[pl.ds(0, K0_CHUNKS)])
    pltpu.sync_copy(dst_hbm.at[2 * s + 1], dst_v.at[pl.ds(K0_CHUNKS, K0_CHUNKS)])

    _zero_rows(rows0, CHUNK, HALF // 16)
    _copy_zero_slice(rows0, acc_sh, ACC_SLC * s, ACC_SLC)
    plsc.subcore_barrier()

    sems = (gsem, ssem)
    pltpu.async_copy(hs_tbl.at[src_v.at[0]], rows[0], sems[0])

    def step(i, _):
        for b in range(2):
            j = 2 * i + b
            nxt = 1 - b

            @pl.when(j + 1 < K2_CHUNKS)
            def _():
                pltpu.async_copy(hs_tbl.at[src_v.at[j + 1]], rows[nxt],
                                 sems[nxt])

            pltpu.make_async_copy(hs_tbl.at[src_v.at[j]], rows[b],
                                  sems[b]).wait()
            pltpu.sync_copy(rows[b], acc_sh.at[dst_v.at[j]], add=True)
        return 0

    lax.fori_loop(0, K2_CHUNKS // 2, step, 0)
    plsc.subcore_barrier()
    pltpu.sync_copy(acc_sh.at[pl.ds(ACC_SLC * s, ACC_SLC)],
                    acc_out.at[c, pl.ds(ACC_SLC * s, ACC_SLC)])


@functools.cache
def _sc_kernels():
    deg_k = pl.kernel(
        _deg_body,
        out_type=jax.ShapeDtypeStruct((2, N_ACC, 16), jnp.float32),
        mesh=_mesh(),
        scratch_types=[
            pltpu.VMEM((K0_CHUNKS, CHUNK), jnp.int32),
            pltpu.VMEM((CHUNK, 16), jnp.float32),
            pltpu.VMEM_SHARED((N_ACC, 16), jnp.float32),
        ],
    )
    gs_k = pl.kernel(
        _gather_scatter_body,
        out_type=jax.ShapeDtypeStruct((2, N_ACC, HALF), jnp.float32),
        mesh=_mesh(),
        scratch_types=[
            pltpu.VMEM((K2_CHUNKS, CHUNK), jnp.int32),
            pltpu.VMEM((K2_CHUNKS, CHUNK), jnp.int32),
            pltpu.VMEM((CHUNK, HALF), jnp.float32),
            pltpu.VMEM((CHUNK, HALF), jnp.float32),
            pltpu.VMEM_SHARED((N_ACC, HALF), jnp.float32),
            pltpu.SemaphoreType.DMA,
            pltpu.SemaphoreType.DMA,
        ],
        compiler_params=pltpu.CompilerParams(use_tc_tiling_on_sc=False),
    )
    return deg_k, gs_k


def _dinv_from(deg_ref):
    deg = deg_ref[0, :, 0:1] + deg_ref[1, :, 0:1] + 1.0
    return lax.rsqrt(deg)


def _layer1_body(deg_ref, x_ref, w_ref, hs_ref):
    dinv = _dinv_from(deg_ref)
    h = jnp.dot(x_ref[...], w_ref[...], preferred_element_type=jnp.float32)
    hs = h * dinv
    hs_ref[0] = hs[:, :HALF]
    hs_ref[1] = hs[:, HALF:]


def _layer2_body(deg_ref, acc_ref, hs1_ref, b1_ref, w_ref, hs2_ref):
    dinv = _dinv_from(deg_ref)
    z0 = dinv * (acc_ref[0] + hs1_ref[0])
    z1 = dinv * (acc_ref[1] + hs1_ref[1])
    z = jnp.concatenate([z0, z1], axis=1) + b1_ref[...]
    z = jnp.maximum(z, 0.0)
    h2 = jnp.dot(z, w_ref[...], preferred_element_type=jnp.float32)
    hs2 = h2 * dinv
    hs2_ref[0] = hs2[:, :HALF]
    hs2_ref[1] = hs2[:, HALF:]


def _final_body(deg_ref, acc_ref, hs2_ref, b2_ref, out_ref):
    dinv = _dinv_from(deg_ref)
    o0 = dinv * (acc_ref[0] + hs2_ref[0])
    o1 = dinv * (acc_ref[1] + hs2_ref[1])
    out_ref[...] = jnp.concatenate([o0, o1], axis=1) + b2_ref[...]


_DEG_SPEC = pl.BlockSpec((2, RBLK, 16), lambda i: (0, i, 0))
_SPLIT_SPEC = pl.BlockSpec((2, RBLK, HALF), lambda i: (0, i, 0))
_FULL_SPEC = pl.BlockSpec((RBLK, D), lambda i: (i, 0))
_W_SPEC = pl.BlockSpec((D, D), lambda i: (0, 0))
_B_SPEC = pl.BlockSpec((1, D), lambda i: (0, 0))

_layer1 = pl.pallas_call(
    _layer1_body,
    grid=(N // RBLK,),
    in_specs=[_DEG_SPEC, _FULL_SPEC, _W_SPEC],
    out_specs=_SPLIT_SPEC,
    out_shape=jax.ShapeDtypeStruct((2, N_ACC, HALF), jnp.float32),
)

_layer2 = pl.pallas_call(
    _layer2_body,
    grid=(N // RBLK,),
    in_specs=[_DEG_SPEC, _SPLIT_SPEC, _SPLIT_SPEC, _B_SPEC, _W_SPEC],
    out_specs=_SPLIT_SPEC,
    out_shape=jax.ShapeDtypeStruct((2, N_ACC, HALF), jnp.float32),
)

_final = pl.pallas_call(
    _final_body,
    grid=(N // RBLK,),
    in_specs=[_DEG_SPEC, _SPLIT_SPEC, _SPLIT_SPEC, _B_SPEC],
    out_specs=_FULL_SPEC,
    out_shape=jax.ShapeDtypeStruct((N, D), jnp.float32),
)


@jax.jit
def _kernel_impl(x, edge_index, W1, b1, W2, b2):
    deg_kernel, gather_scatter_kernel = _sc_kernels()
    src = edge_index[0].astype(jnp.int32)
    dst = edge_index[1].astype(jnp.int32)
    pad = E_PAD - E
    src = jnp.concatenate([src, jnp.zeros((pad,), jnp.int32)])
    dst = jnp.concatenate([dst, jnp.full((pad,), DUMMY, jnp.int32)])
    src = src.reshape(32, K0_CHUNKS, CHUNK)
    dst = dst.reshape(32, K0_CHUNKS, CHUNK)

    deg = deg_kernel(dst)
    hs1 = _layer1(deg, x, W1)
    acc1 = gather_scatter_kernel(hs1, src, dst)
    hs2 = _layer2(deg, acc1, hs1, b1.reshape(1, D), W2)
    acc2 = gather_scatter_kernel(hs2, src, dst)
    return _final(deg, acc2, hs2, b2.reshape(1, D))


def kernel(x, edge_index, W1, b1, W2, b2):
    return _kernel_impl(x, edge_index, W1, b1, W2, b2)


# speedup vs baseline: 1.5081x; 1.0037x over previous
import functools

import jax
import jax.numpy as jnp
from jax import lax
from jax.experimental import pallas as pl
from jax.experimental.pallas import tpu as pltpu
from jax.experimental.pallas import tpu_sc as plsc

N = 10000
D = 128
HALF = 64
E = 320000
CHUNK = 128
K0_CHUNKS = 79
E_PAD = 32 * K0_CHUNKS * CHUNK
K2_CHUNKS = 158
E_PAD2 = 16 * K2_CHUNKS * CHUNK
DUMMY = N
N_ACC = 10112
ACC_SLC = N_ACC // 16
RBLK = 1000


def _mesh():
    return plsc.VectorSubcoreMesh(core_axis_name="c", subcore_axis_name="s",
                                  num_cores=2, num_subcores=16)


def _zero_rows(ref, nrows, ncols16):
    zero = jnp.zeros((16,), jnp.float32)

    def body(t, _):
        i = t // ncols16
        k = t % ncols16
        ref[i, pl.ds(k * 16, 16)] = zero
        return 0

    lax.fori_loop(0, nrows * ncols16, body, 0)


def _copy_zero_slice(zsrc, shared, base, nrows):
    full, rem = nrows // CHUNK, nrows % CHUNK
    for k in range(full):
        pltpu.sync_copy(zsrc, shared.at[pl.ds(base + k * CHUNK, CHUNK)])
    if rem:
        pltpu.sync_copy(zsrc.at[pl.ds(0, rem)],
                        shared.at[pl.ds(base + full * CHUNK, rem)])


def _deg_body(dst_hbm, deg_out, dst_v, ones_v, deg_sh):
    c = lax.axis_index("c")
    s = lax.axis_index("s")
    tid = c * 16 + s
    pltpu.sync_copy(dst_hbm.at[tid], dst_v)

    _zero_rows(ones_v, CHUNK, 1)
    _copy_zero_slice(ones_v, deg_sh, ACC_SLC * s, ACC_SLC)

    one = jnp.full((16,), 1.0, jnp.float32)

    def fill(i, _):
        ones_v[i, :] = one
        return 0

    lax.fori_loop(0, CHUNK, fill, 0)
    plsc.subcore_barrier()

    def body(j, _):
        pltpu.sync_copy(ones_v, deg_sh.at[dst_v.at[j]], add=True)
        return 0

    lax.fori_loop(0, K0_CHUNKS, body, 0)
    plsc.subcore_barrier()
    pltpu.sync_copy(deg_sh.at[pl.ds(ACC_SLC * s, ACC_SLC)],
                    deg_out.at[c, pl.ds(ACC_SLC * s, ACC_SLC)])


def _gather_scatter_body(hs_hbm, src_hbm, dst_hbm, acc_out,
                           src_v, dst_v, rows0, rows1, acc_sh, sem0, sem1):
    rows = (rows0, rows1)
    sems = (sem0, sem1)
    c = lax.axis_index("c")
    s = lax.axis_index("s")
    hs_tbl = hs_hbm.at[c]

    pltpu.sync_copy(src_hbm.at[s], src_v)
    pltpu.sync_copy(dst_hbm.at[s], dst_v)

    _zero_rows(rows0, CHUNK, HALF // 16)
    _copy_zero_slice(rows0, acc_sh, ACC_SLC * s, ACC_SLC)
    plsc.subcore_barrier()

    pltpu.async_copy(hs_tbl.at[src_v.at[0]], rows[0], sems[0])

    def step(i, _):
        for b in range(2):
            j = 2 * i + b
            nxt = 1 - b

            @pl.when(j + 1 < K2_CHUNKS)
            def _():
                pltpu.async_copy(hs_tbl.at[src_v.at[j + 1]], rows[nxt],
                                 sems[nxt])

            pltpu.make_async_copy(hs_tbl.at[src_v.at[j]], rows[b],
                                  sems[b]).wait()
            pltpu.sync_copy(rows[b], acc_sh.at[dst_v.at[j]], add=True)
        return 0

    lax.fori_loop(0, K2_CHUNKS // 2, step, 0)
    plsc.subcore_barrier()
    pltpu.sync_copy(acc_sh.at[pl.ds(ACC_SLC * s, ACC_SLC)],
                    acc_out.at[c, pl.ds(ACC_SLC * s, ACC_SLC)])


@functools.cache
def _sc_kernels():
    deg_k = pl.kernel(
        _deg_body,
        out_type=jax.ShapeDtypeStruct((2, N_ACC, 16), jnp.float32),
        mesh=_mesh(),
        scratch_types=[
            pltpu.VMEM((K0_CHUNKS, CHUNK), jnp.int32),
            pltpu.VMEM((CHUNK, 16), jnp.float32),
            pltpu.VMEM_SHARED((N_ACC, 16), jnp.float32),
        ],
    )
    gs_k = pl.kernel(
        _gather_scatter_body,
        out_type=jax.ShapeDtypeStruct((2, N_ACC, HALF), jnp.float32),
        mesh=_mesh(),
        scratch_types=[
            pltpu.VMEM((K2_CHUNKS, CHUNK), jnp.int32),
            pltpu.VMEM((K2_CHUNKS, CHUNK), jnp.int32),
            pltpu.VMEM((CHUNK, HALF), jnp.float32),
            pltpu.VMEM((CHUNK, HALF), jnp.float32),
            pltpu.VMEM_SHARED((N_ACC, HALF), jnp.float32),
            pltpu.SemaphoreType.DMA,
            pltpu.SemaphoreType.DMA,
        ],
        compiler_params=pltpu.CompilerParams(use_tc_tiling_on_sc=False),
    )
    return deg_k, gs_k


def _dinv_from(deg_ref):
    deg = deg_ref[0, :, 0:1] + deg_ref[1, :, 0:1] + 1.0
    return lax.rsqrt(deg)


def _layer1_body(deg_ref, x_ref, w_ref, hs_ref):
    dinv = _dinv_from(deg_ref)
    h = jnp.dot(x_ref[...], w_ref[...], preferred_element_type=jnp.float32)
    hs = h * dinv
    hs_ref[0] = hs[:, :HALF]
    hs_ref[1] = hs[:, HALF:]


def _layer2_body(deg_ref, acc_ref, hs1_ref, b1_ref, w_ref, hs2_ref):
    dinv = _dinv_from(deg_ref)
    z0 = dinv * (acc_ref[0] + hs1_ref[0])
    z1 = dinv * (acc_ref[1] + hs1_ref[1])
    z = jnp.concatenate([z0, z1], axis=1) + b1_ref[...]
    z = jnp.maximum(z, 0.0)
    h2 = jnp.dot(z, w_ref[...], preferred_element_type=jnp.float32)
    hs2 = h2 * dinv
    hs2_ref[0] = hs2[:, :HALF]
    hs2_ref[1] = hs2[:, HALF:]


def _final_body(deg_ref, acc_ref, hs2_ref, b2_ref, out_ref):
    dinv = _dinv_from(deg_ref)
    o0 = dinv * (acc_ref[0] + hs2_ref[0])
    o1 = dinv * (acc_ref[1] + hs2_ref[1])
    out_ref[...] = jnp.concatenate([o0, o1], axis=1) + b2_ref[...]


_DEG_SPEC = pl.BlockSpec((2, RBLK, 16), lambda i: (0, i, 0))
_SPLIT_SPEC = pl.BlockSpec((2, RBLK, HALF), lambda i: (0, i, 0))
_FULL_SPEC = pl.BlockSpec((RBLK, D), lambda i: (i, 0))
_W_SPEC = pl.BlockSpec((D, D), lambda i: (0, 0))
_B_SPEC = pl.BlockSpec((1, D), lambda i: (0, 0))

_layer1 = pl.pallas_call(
    _layer1_body,
    grid=(N // RBLK,),
    in_specs=[_DEG_SPEC, _FULL_SPEC, _W_SPEC],
    out_specs=_SPLIT_SPEC,
    out_shape=jax.ShapeDtypeStruct((2, N_ACC, HALF), jnp.float32),
)

_layer2 = pl.pallas_call(
    _layer2_body,
    grid=(N // RBLK,),
    in_specs=[_DEG_SPEC, _SPLIT_SPEC, _SPLIT_SPEC, _B_SPEC, _W_SPEC],
    out_specs=_SPLIT_SPEC,
    out_shape=jax.ShapeDtypeStruct((2, N_ACC, HALF), jnp.float32),
)

_final = pl.pallas_call(
    _final_body,
    grid=(N // RBLK,),
    in_specs=[_DEG_SPEC, _SPLIT_SPEC, _SPLIT_SPEC, _B_SPEC],
    out_specs=_FULL_SPEC,
    out_shape=jax.ShapeDtypeStruct((N, D), jnp.float32),
)


@jax.jit
def _kernel_impl(x, edge_index, W1, b1, W2, b2):
    deg_kernel, gather_scatter_kernel = _sc_kernels()
    src = edge_index[0].astype(jnp.int32)
    dst = edge_index[1].astype(jnp.int32)
    dst_deg = jnp.concatenate(
        [dst, jnp.full((E_PAD - E,), DUMMY, jnp.int32)]).reshape(
            32, K0_CHUNKS, CHUNK)
    src_gs = jnp.concatenate(
        [src, jnp.zeros((E_PAD2 - E,), jnp.int32)]).reshape(
            16, K2_CHUNKS, CHUNK)
    dst_gs = jnp.concatenate(
        [dst, jnp.full((E_PAD2 - E,), DUMMY, jnp.int32)]).reshape(
            16, K2_CHUNKS, CHUNK)

    deg = deg_kernel(dst_deg)
    hs1 = _layer1(deg, x, W1)
    acc1 = gather_scatter_kernel(hs1, src_gs, dst_gs)
    hs2 = _layer2(deg, acc1, hs1, b1.reshape(1, D), W2)
    acc2 = gather_scatter_kernel(hs2, src_gs, dst_gs)
    return _final(deg, acc2, hs2, b2.reshape(1, D))


def kernel(x, edge_index, W1, b1, W2, b2):
    return _kernel_impl(x, edge_index, W1, b1, W2, b2)
